# speedup vs baseline: 1.2548x; 1.2548x over previous
import functools

import jax
import jax.numpy as jnp
from jax.experimental import pallas as pl
from jax.experimental.pallas import tpu as pltpu

BATCH = 1024
UNITS = 2048
D_IN = 512
BATCH_TILE = 256


def _cell_body(x_ref, ps_ref, k_ref, r_ref, b_ref, o_ref):
    ip = jnp.dot(x_ref[...], k_ref[...], preferred_element_type=jnp.float32)
    sp = jnp.dot(ps_ref[...], r_ref[...], preferred_element_type=jnp.float32)
    o_ref[...] = jnp.tanh(ip + sp + b_ref[...])


def kernel(inputs, prev_state, kernel, recurrent_kernel, bias):
    bias2 = bias.reshape(1, UNITS)
    grid = (BATCH // BATCH_TILE,)
    out = pl.pallas_call(
        _cell_body,
        grid=grid,
        in_specs=[
            pl.BlockSpec((BATCH_TILE, D_IN), lambda i: (i, 0)),
            pl.BlockSpec((BATCH_TILE, UNITS), lambda i: (i, 0)),
            pl.BlockSpec((D_IN, UNITS), lambda i: (0, 0)),
            pl.BlockSpec((UNITS, UNITS), lambda i: (0, 0)),
            pl.BlockSpec((1, UNITS), lambda i: (0, 0)),
        ],
        out_specs=pl.BlockSpec((BATCH_TILE, UNITS), lambda i: (i, 0)),
        out_shape=jax.ShapeDtypeStruct((BATCH, UNITS), jnp.float32),
        compiler_params=pltpu.CompilerParams(
            dimension_semantics=("arbitrary",),
        ),
    )(inputs, prev_state, kernel, recurrent_kernel, bias2)
    return out
